# baseline (speedup 1.0000x reference)
import functools

import jax
import jax.numpy as jnp
from jax import lax
from jax.experimental import pallas as pl
from jax.experimental.pallas import tpu as pltpu
from jax.experimental.pallas import tpu_sc as plsc

BLK_M = 1024
BLK_K = 2048
_DN = (((1,), (1,)), ((), ()))


def _norm_body(x_ref, c_ref, cc_ref):
    x = x_ref[...]
    n = jnp.sqrt(jnp.sum(x * x, axis=1, keepdims=True))
    c = x / (n + 1e-8)
    c_ref[...] = c
    cc_ref[...] = jnp.sum(c * c, axis=1, keepdims=True)


def _normalize(centers):
    K, D = centers.shape
    blk = 1024
    return pl.pallas_call(
        _norm_body,
        grid=(K // blk,),
        in_specs=[pl.BlockSpec((blk, D), lambda i: (i, 0))],
        out_specs=[
            pl.BlockSpec((blk, D), lambda i: (i, 0)),
            pl.BlockSpec((blk, 1), lambda i: (i, 0)),
        ],
        out_shape=[
            jax.ShapeDtypeStruct((K, D), jnp.float32),
            jax.ShapeDtypeStruct((K, 1), jnp.float32),
        ],
    )(centers)


def _dist_body(l_ref, c_ref, cct_ref, zz_ref, codes_ref, minv_s, mini_s):
    k = pl.program_id(1)
    nk = pl.num_programs(1)

    @pl.when(k == 0)
    def _():
        minv_s[...] = jnp.full(minv_s.shape, jnp.inf, jnp.float32)
        mini_s[...] = jnp.zeros(mini_s.shape, jnp.int32)

    cx = c_ref[...]
    hi = cx.astype(jnp.bfloat16)
    lo = (cx - hi.astype(jnp.float32)).astype(jnp.bfloat16)
    l = l_ref[...]
    m = (lax.dot_general(l, hi, _DN, preferred_element_type=jnp.float32)
         + lax.dot_general(l, lo, _DN, preferred_element_type=jnp.float32))
    scores = (zz_ref[...] - m) + cct_ref[...]
    lmin = jnp.min(scores, axis=1, keepdims=True)
    larg = (jnp.argmin(scores, axis=1).astype(jnp.int32) + k * BLK_K)[:, None]
    better = lmin < minv_s[...]
    minv_s[...] = jnp.where(better, lmin, minv_s[...])
    mini_s[...] = jnp.where(better, larg, mini_s[...])

    @pl.when(k == nk - 1)
    def _():
        codes_ref[...] = mini_s[...]


def _zz_body(z_ref, o_ref):
    t = z_ref[...]
    o_ref[...] = jnp.sum(t * t, axis=1, keepdims=True)


def _assign(flat, c, cct):
    M, D = flat.shape
    K = c.shape[0]
    zz = pl.pallas_call(
        _zz_body,
        grid=(M // BLK_M,),
        in_specs=[pl.BlockSpec((BLK_M, D), lambda m: (m, 0))],
        out_specs=pl.BlockSpec((BLK_M, 1), lambda m: (m, 0)),
        out_shape=jax.ShapeDtypeStruct((M, 1), jnp.float32),
    )(flat)
    lhs = (2.0 * flat).astype(jnp.bfloat16)
    return pl.pallas_call(
        _dist_body,
        grid=(M // BLK_M, K // BLK_K),
        in_specs=[
            pl.BlockSpec((BLK_M, D), lambda m, k: (m, 0)),
            pl.BlockSpec((BLK_K, D), lambda m, k: (k, 0)),
            pl.BlockSpec((1, BLK_K), lambda m, k: (0, k)),
            pl.BlockSpec((BLK_M, 1), lambda m, k: (m, 0)),
        ],
        out_specs=pl.BlockSpec((BLK_M, 1), lambda m, k: (m, 0)),
        out_shape=jax.ShapeDtypeStruct((M, 1), jnp.int32),
        scratch_shapes=[
            pltpu.VMEM((BLK_M, 1), jnp.float32),
            pltpu.VMEM((BLK_M, 1), jnp.int32),
        ],
        compiler_params=pltpu.CompilerParams(
            dimension_semantics=("arbitrary", "arbitrary"),
        ),
    )(lhs, c, cct, zz)


def _gather(c, codes2d):
    K, D = c.shape
    B = codes2d.shape[0] * codes2d.shape[1]
    info = plsc.get_sparse_core_info()
    nw = info.num_cores * info.num_subcores
    b_per_w = B // nw
    chunk = 128
    nchunks = b_per_w // chunk
    mesh = plsc.VectorSubcoreMesh(core_axis_name="c", subcore_axis_name="s")

    @functools.partial(
        pl.kernel,
        mesh=mesh,
        out_type=jax.ShapeDtypeStruct((B, D), jnp.float32),
        scratch_types=[
            pltpu.VMEM((nchunks, chunk), jnp.int32),
            pltpu.VMEM((b_per_w, D), jnp.float32),
            pltpu.SemaphoreType.DMA,
        ],
        compiler_params=pltpu.CompilerParams(use_tc_tiling_on_sc=False),
    )
    def gather_k(table_hbm, idx_hbm, out_hbm, idx_v, rows_v, sem):
        wid = lax.axis_index("s") * info.num_cores + lax.axis_index("c")
        base = wid * b_per_w
        pltpu.sync_copy(idx_hbm.at[pl.ds(wid * nchunks, nchunks)], idx_v)
        for j in range(nchunks):
            pltpu.async_copy(
                table_hbm.at[idx_v.at[j]],
                rows_v.at[pl.ds(j * chunk, chunk)],
                sem,
            )
        for j in range(nchunks):
            pltpu.make_async_copy(
                table_hbm.at[idx_v.at[j]],
                rows_v.at[pl.ds(j * chunk, chunk)],
                sem,
            ).wait()
        pltpu.sync_copy(rows_v, out_hbm.at[pl.ds(base, b_per_w)])

    return gather_k(c, codes2d)


def kernel(z, centers):
    B, T, D = z.shape
    c, cc_col = _normalize(centers)
    cct = cc_col.reshape(1, centers.shape[0])
    flat = z.reshape(B * T, D)
    codes = _assign(flat, c, cct).reshape(B * T // 128, 128)
    q = _gather(c, codes)
    return q.reshape(B, T, D)


# speedup vs baseline: 1.2574x; 1.2574x over previous
import functools

import jax
import jax.numpy as jnp
from jax import lax
from jax.experimental import pallas as pl
from jax.experimental.pallas import tpu as pltpu
from jax.experimental.pallas import tpu_sc as plsc

BLK_M = 1024
BLK_K = 2048
_DN = (((1,), (1,)), ((), ()))


def _norm_body(x_ref, c_ref, cc_ref):
    x = x_ref[...]
    n = jnp.sqrt(jnp.sum(x * x, axis=1, keepdims=True))
    c = x / (n + 1e-8)
    c_ref[...] = c
    cc_ref[...] = jnp.sum(c * c, axis=1, keepdims=True)


def _normalize(centers):
    K, D = centers.shape
    blk = 1024
    return pl.pallas_call(
        _norm_body,
        grid=(K // blk,),
        in_specs=[pl.BlockSpec((blk, D), lambda i: (i, 0))],
        out_specs=[
            pl.BlockSpec((blk, D), lambda i: (i, 0)),
            pl.BlockSpec((blk, 1), lambda i: (i, 0)),
        ],
        out_shape=[
            jax.ShapeDtypeStruct((K, D), jnp.float32),
            jax.ShapeDtypeStruct((K, 1), jnp.float32),
        ],
    )(centers)


def _dist_body(l_ref, c_ref, cct_ref, zz_ref, codes_ref, minv_s, mini_s):
    k = pl.program_id(1)
    nk = pl.num_programs(1)

    @pl.when(k == 0)
    def _():
        minv_s[...] = jnp.full(minv_s.shape, jnp.inf, jnp.float32)
        mini_s[...] = jnp.zeros(mini_s.shape, jnp.int32)

    m = lax.dot_general(l_ref[...], c_ref[...], _DN,
                        preferred_element_type=jnp.float32)
    scores = (zz_ref[...] - m) + cct_ref[...]
    lmin = jnp.min(scores, axis=1, keepdims=True)
    larg = (jnp.argmin(scores, axis=1).astype(jnp.int32) + k * BLK_K)[:, None]
    better = lmin < minv_s[...]
    minv_s[...] = jnp.where(better, lmin, minv_s[...])
    mini_s[...] = jnp.where(better, larg, mini_s[...])

    @pl.when(k == nk - 1)
    def _():
        codes_ref[...] = mini_s[...]


def _zz_body(z_ref, o_ref):
    t = z_ref[...]
    o_ref[...] = jnp.sum(t * t, axis=1, keepdims=True)


def _assign(flat, c, cct):
    M, D = flat.shape
    K = c.shape[0]
    zz = pl.pallas_call(
        _zz_body,
        grid=(M // BLK_M,),
        in_specs=[pl.BlockSpec((BLK_M, D), lambda m: (m, 0))],
        out_specs=pl.BlockSpec((BLK_M, 1), lambda m: (m, 0)),
        out_shape=jax.ShapeDtypeStruct((M, 1), jnp.float32),
    )(flat)
    lhs = (2.0 * flat).astype(jnp.bfloat16)
    return pl.pallas_call(
        _dist_body,
        grid=(M // BLK_M, K // BLK_K),
        in_specs=[
            pl.BlockSpec((BLK_M, D), lambda m, k: (m, 0)),
            pl.BlockSpec((BLK_K, D), lambda m, k: (k, 0)),
            pl.BlockSpec((1, BLK_K), lambda m, k: (0, k)),
            pl.BlockSpec((BLK_M, 1), lambda m, k: (m, 0)),
        ],
        out_specs=pl.BlockSpec((BLK_M, 1), lambda m, k: (m, 0)),
        out_shape=jax.ShapeDtypeStruct((M, 1), jnp.int32),
        scratch_shapes=[
            pltpu.VMEM((BLK_M, 1), jnp.float32),
            pltpu.VMEM((BLK_M, 1), jnp.int32),
        ],
        compiler_params=pltpu.CompilerParams(
            dimension_semantics=("arbitrary", "arbitrary"),
        ),
    )(lhs, c, cct, zz)


def _gather(c, codes2d):
    K, D = c.shape
    B = codes2d.shape[0] * codes2d.shape[1]
    info = plsc.get_sparse_core_info()
    nw = info.num_cores * info.num_subcores
    b_per_w = B // nw
    chunk = 128
    nchunks = b_per_w // chunk
    mesh = plsc.VectorSubcoreMesh(core_axis_name="c", subcore_axis_name="s")

    @functools.partial(
        pl.kernel,
        mesh=mesh,
        out_type=jax.ShapeDtypeStruct((B, D), jnp.float32),
        scratch_types=[
            pltpu.VMEM((nchunks, chunk), jnp.int32),
            pltpu.VMEM((b_per_w, D), jnp.float32),
            pltpu.SemaphoreType.DMA,
        ],
        compiler_params=pltpu.CompilerParams(use_tc_tiling_on_sc=False),
    )
    def gather_k(table_hbm, idx_hbm, out_hbm, idx_v, rows_v, sem):
        wid = lax.axis_index("s") * info.num_cores + lax.axis_index("c")
        base = wid * b_per_w
        pltpu.sync_copy(idx_hbm.at[pl.ds(wid * nchunks, nchunks)], idx_v)
        for j in range(nchunks):
            pltpu.async_copy(
                table_hbm.at[idx_v.at[j]],
                rows_v.at[pl.ds(j * chunk, chunk)],
                sem,
            )
        for j in range(nchunks):
            pltpu.make_async_copy(
                table_hbm.at[idx_v.at[j]],
                rows_v.at[pl.ds(j * chunk, chunk)],
                sem,
            ).wait()
        pltpu.sync_copy(rows_v, out_hbm.at[pl.ds(base, b_per_w)])

    return gather_k(c, codes2d)


def kernel(z, centers):
    B, T, D = z.shape
    c, cc_col = _normalize(centers)
    cct = cc_col.reshape(1, centers.shape[0])
    flat = z.reshape(B * T, D)
    codes = _assign(flat, c, cct).reshape(B * T // 128, 128)
    q = _gather(c, codes)
    return q.reshape(B, T, D)
